# initial kernel scaffold (unmeasured)
import jax
import jax.numpy as jnp
from jax import lax
from jax.experimental import pallas as pl
from jax.experimental.pallas import tpu as pltpu

N_DEV = 4
N_EXP = 16
EXP_PER_DEV = N_EXP // N_DEV
CAP = 409


def kernel(x, router_W, route_idx, expert_W):
    n_tok, d_model = x.shape
    _, _, d_ff = expert_W.shape

    def body(x_ref, route_ref, ew_ref, out_ref,
             gw_ref, gidx_ref, w_send, w_recv, i_send, i_recv):
        my = lax.axis_index("i")
        left = lax.rem(my + N_DEV - 1, N_DEV)
        right = lax.rem(my + 1, N_DEV)

        barrier = pltpu.get_barrier_semaphore()
        for nbr in (left, right):
            pl.semaphore_signal(barrier, inc=1, device_id=(nbr,),
                                device_id_type=pl.DeviceIdType.MESH)
        pl.semaphore_wait(barrier, 2)

        gw_ref[pl.ds(my * EXP_PER_DEV, EXP_PER_DEV)] = (
            ew_ref[...].astype(jnp.bfloat16))
        gidx_ref[pl.ds(my, 1)] = route_ref[...].reshape(1, n_tok, 1)

        for h in range(N_DEV - 1):
            blk = lax.rem(my - h + N_DEV, N_DEV)
            w_rdma = pltpu.make_async_remote_copy(
                src_ref=gw_ref.at[pl.ds(blk * EXP_PER_DEV, EXP_PER_DEV)],
                dst_ref=gw_ref.at[pl.ds(blk * EXP_PER_DEV, EXP_PER_DEV)],
                send_sem=w_send.at[h],
                recv_sem=w_recv.at[h],
                device_id=(right,),
                device_id_type=pl.DeviceIdType.MESH,
            )
            i_rdma = pltpu.make_async_remote_copy(
                src_ref=gidx_ref.at[pl.ds(blk, 1)],
                dst_ref=gidx_ref.at[pl.ds(blk, 1)],
                send_sem=i_send.at[h],
                recv_sem=i_recv.at[h],
                device_id=(right,),
                device_id_type=pl.DeviceIdType.MESH,
            )
            w_rdma.start()
            i_rdma.start()
            w_rdma.wait()
            i_rdma.wait()

        gidx = gidx_ref[...]
        e3 = lax.broadcasted_iota(jnp.int32, (N_DEV, n_tok, N_EXP), 2)
        m_all = (gidx == e3).astype(jnp.int32)
        counts = jnp.sum(m_all, axis=1)
        j = lax.broadcasted_iota(jnp.int32, (N_DEV, N_EXP), 0)
        prefix = jnp.sum(jnp.where(j < my, counts, 0),
                         axis=0, keepdims=True)

        eloc = lax.broadcasted_iota(jnp.int32, (n_tok, N_EXP), 1)
        m_loc = (route_ref[...] == eloc).astype(jnp.int32)
        rank_excl = jnp.cumsum(m_loc, axis=0) - m_loc
        kept = (m_loc > 0) & ((prefix + rank_excl) < CAP)

        x_bf = x_ref[...].astype(jnp.bfloat16)
        acc = jnp.zeros((n_tok, d_ff), jnp.float32)
        for e in range(N_EXP):
            xe = jnp.where(kept[:, e:e + 1], x_bf, 0)
            acc = acc + jnp.dot(xe, gw_ref[e],
                                preferred_element_type=jnp.float32)
        out_ref[...] = acc

    return pl.pallas_call(
        body,
        out_shape=jax.ShapeDtypeStruct((n_tok, d_ff), jnp.float32),
        in_specs=[
            pl.BlockSpec(memory_space=pltpu.VMEM),
            pl.BlockSpec(memory_space=pltpu.VMEM),
            pl.BlockSpec(memory_space=pltpu.VMEM),
        ],
        out_specs=pl.BlockSpec(memory_space=pltpu.VMEM),
        scratch_shapes=[
            pltpu.VMEM((N_EXP, d_model, d_ff), jnp.bfloat16),
            pltpu.VMEM((N_DEV, n_tok, 1), jnp.int32),
            pltpu.SemaphoreType.DMA((N_DEV - 1,)),
            pltpu.SemaphoreType.DMA((N_DEV - 1,)),
            pltpu.SemaphoreType.DMA((N_DEV - 1,)),
            pltpu.SemaphoreType.DMA((N_DEV - 1,)),
        ],
        compiler_params=pltpu.CompilerParams(collective_id=0),
    )(x, route_idx, expert_W)


# baseline (device time: 206290 ns/iter reference)
import jax
import jax.numpy as jnp
from jax import lax
from jax.experimental import pallas as pl
from jax.experimental.pallas import tpu as pltpu

N_DEV = 4
N_EXP = 16
EXP_PER_DEV = N_EXP // N_DEV
CAP = 409
LANES = 128


def kernel(x, router_W, route_idx, expert_W):
    n_tok, d_model = x.shape
    _, _, d_ff = expert_W.shape

    def body(x_ref, route_ref, ew_ref, out_ref,
             gw_ref, gcnt_ref, w_send, w_recv, c_send, c_recv):
        my = lax.axis_index("i")
        left = lax.rem(my + N_DEV - 1, N_DEV)
        right = lax.rem(my + 1, N_DEV)

        barrier = pltpu.get_barrier_semaphore()
        for nbr in (left, right):
            pl.semaphore_signal(barrier, inc=1, device_id=(nbr,),
                                device_id_type=pl.DeviceIdType.MESH)
        pl.semaphore_wait(barrier, 2)

        eloc = lax.broadcasted_iota(jnp.int32, (n_tok, N_EXP), 1)
        m_loc = (route_ref[...] == eloc).astype(jnp.int32)
        cnt = jnp.sum(m_loc, axis=0, keepdims=True)

        gw_ref[pl.ds(my * EXP_PER_DEV, EXP_PER_DEV)] = ew_ref[...]
        gcnt_ref[pl.ds(my, 1)] = jnp.zeros((1, 1, LANES), jnp.int32)
        gcnt_ref[pl.ds(my, 1), 0:1, 0:N_EXP] = cnt.reshape(1, 1, N_EXP)

        for h in range(N_DEV - 1):
            blk = lax.rem(my - h + N_DEV, N_DEV)
            w_rdma = pltpu.make_async_remote_copy(
                src_ref=gw_ref.at[pl.ds(blk * EXP_PER_DEV, EXP_PER_DEV)],
                dst_ref=gw_ref.at[pl.ds(blk * EXP_PER_DEV, EXP_PER_DEV)],
                send_sem=w_send.at[h],
                recv_sem=w_recv.at[h],
                device_id=(right,),
                device_id_type=pl.DeviceIdType.MESH,
            )
            c_rdma = pltpu.make_async_remote_copy(
                src_ref=gcnt_ref.at[pl.ds(blk, 1)],
                dst_ref=gcnt_ref.at[pl.ds(blk, 1)],
                send_sem=c_send.at[h],
                recv_sem=c_recv.at[h],
                device_id=(right,),
                device_id_type=pl.DeviceIdType.MESH,
            )
            w_rdma.start()
            c_rdma.start()
            w_rdma.wait()
            c_rdma.wait()

        gcnt = gcnt_ref[...]
        j = lax.broadcasted_iota(jnp.int32, (N_DEV, 1, LANES), 0)
        prefix = jnp.sum(jnp.where(j < my, gcnt, 0), axis=0)
        prefix16 = prefix[0:1, 0:N_EXP]

        c = m_loc
        s = 1
        while s < n_tok:
            c = c + jnp.concatenate(
                [jnp.zeros((s, N_EXP), jnp.int32), c[:-s, :]], axis=0)
            s *= 2
        rank_excl = c - m_loc
        kept = (m_loc > 0) & ((prefix16 + rank_excl) < CAP)

        x_bf = x_ref[...]
        acc = jnp.zeros((n_tok, d_ff), jnp.float32)
        for e in range(N_EXP):
            xe = jnp.where(kept[:, e:e + 1], x_bf, 0)
            acc = acc + jnp.dot(xe, gw_ref[e],
                                preferred_element_type=jnp.float32)
        out_ref[...] = acc

    return pl.pallas_call(
        body,
        out_shape=jax.ShapeDtypeStruct((n_tok, d_ff), jnp.float32),
        in_specs=[
            pl.BlockSpec(memory_space=pltpu.VMEM),
            pl.BlockSpec(memory_space=pltpu.VMEM),
            pl.BlockSpec(memory_space=pltpu.VMEM),
        ],
        out_specs=pl.BlockSpec(memory_space=pltpu.VMEM),
        scratch_shapes=[
            pltpu.VMEM((N_EXP, d_model, d_ff), jnp.bfloat16),
            pltpu.VMEM((N_DEV, 1, LANES), jnp.int32),
            pltpu.SemaphoreType.DMA((N_DEV - 1,)),
            pltpu.SemaphoreType.DMA((N_DEV - 1,)),
            pltpu.SemaphoreType.DMA((N_DEV - 1,)),
            pltpu.SemaphoreType.DMA((N_DEV - 1,)),
        ],
        compiler_params=pltpu.CompilerParams(
            collective_id=0,
            vmem_limit_bytes=100 * 1024 * 1024,
        ),
    )(x.astype(jnp.bfloat16), route_idx, expert_W.astype(jnp.bfloat16))


# device time: 109443 ns/iter; 1.8849x vs baseline; 1.8849x over previous
import jax
import jax.numpy as jnp
from jax import lax
from jax.experimental import pallas as pl
from jax.experimental.pallas import tpu as pltpu

N_DEV = 4
N_EXP = 16
EXP_PER_DEV = N_EXP // N_DEV
HALF = EXP_PER_DEV // 2
CAP = 409
LANES = 128


def kernel(x, router_W, route_idx, expert_W):
    n_tok, d_model = x.shape
    _, _, d_ff = expert_W.shape

    def body(x_ref, route_ref, ew_ref, out_ref,
             gw_ref, gcnt_ref, w_send, w_recv, c_send, c_recv):
        my = lax.axis_index("i")
        left = lax.rem(my + N_DEV - 1, N_DEV)
        right = lax.rem(my + 1, N_DEV)
        diag = lax.rem(my + 2, N_DEV)

        eloc = lax.broadcasted_iota(jnp.int32, (n_tok, N_EXP), 1)
        m_loc = (route_ref[...] == eloc).astype(jnp.int32)
        cnt = jnp.sum(m_loc, axis=0, keepdims=True)

        gw_ref[pl.ds(my * EXP_PER_DEV, EXP_PER_DEV)] = ew_ref[...]
        gcnt_ref[pl.ds(my, 1)] = jnp.zeros((1, 1, LANES), jnp.int32)
        gcnt_ref[pl.ds(my, 1), 0:1, 0:N_EXP] = cnt.reshape(1, 1, N_EXP)

        barrier = pltpu.get_barrier_semaphore()
        for nbr in (left, right, diag):
            pl.semaphore_signal(barrier, inc=1, device_id=(nbr,),
                                device_id_type=pl.DeviceIdType.MESH)
        pl.semaphore_wait(barrier, N_DEV - 1)

        def wcopy(src_lo, n_exp, target, ch):
            return pltpu.make_async_remote_copy(
                src_ref=gw_ref.at[pl.ds(src_lo, n_exp)],
                dst_ref=gw_ref.at[pl.ds(src_lo, n_exp)],
                send_sem=w_send.at[ch],
                recv_sem=w_recv.at[ch],
                device_id=(target,),
                device_id_type=pl.DeviceIdType.MESH,
            )

        def ccopy(target, ch):
            return pltpu.make_async_remote_copy(
                src_ref=gcnt_ref.at[pl.ds(my, 1)],
                dst_ref=gcnt_ref.at[pl.ds(my, 1)],
                send_sem=c_send.at[ch],
                recv_sem=c_recv.at[ch],
                device_id=(target,),
                device_id_type=pl.DeviceIdType.MESH,
            )

        c_to_r = ccopy(right, 0)
        c_to_l = ccopy(left, 1)
        c_to_d = ccopy(diag, 2)
        c_to_r.start()
        c_to_l.start()
        c_to_d.start()

        wa_r = wcopy(my * EXP_PER_DEV, EXP_PER_DEV, right, 0)
        wa_l = wcopy(my * EXP_PER_DEV, EXP_PER_DEV, left, 1)
        wa_r.start()
        wa_l.start()

        c = m_loc
        s = 1
        while s < n_tok:
            c = c + jnp.concatenate(
                [jnp.zeros((s, N_EXP), jnp.int32), c[:-s, :]], axis=0)
            s *= 2
        rank_excl = c - m_loc

        c_to_r.wait_recv()
        c_to_l.wait_recv()
        c_to_d.wait_recv()
        gcnt = gcnt_ref[...]
        j = lax.broadcasted_iota(jnp.int32, (N_DEV, 1, LANES), 0)
        prefix = jnp.sum(jnp.where(j < my, gcnt, 0), axis=0)
        prefix16 = prefix[0:1, 0:N_EXP]

        kept = (m_loc > 0) & ((prefix16 + rank_excl) < CAP)
        kept_tok = jnp.sum(kept, axis=1, keepdims=True) > 0

        x_bf = x_ref[...]
        route = route_ref[...]

        def block_partial(blk):
            acc = None
            for k in range(EXP_PER_DEV):
                e = blk * EXP_PER_DEV + k
                sel = (route == e) & kept_tok
                xe = jnp.where(sel, x_bf, jnp.bfloat16(0))
                w = gw_ref[pl.ds(e, 1)][0]
                p = jnp.dot(xe, w, preferred_element_type=jnp.float32)
                acc = p if acc is None else acc + p
            return acc

        def compute_block(blk):
            out_ref[...] += block_partial(blk)

        out_ref[...] = block_partial(my)

        wa_r.wait_recv()
        wa_l.wait_recv()
        wb_r = wcopy(left * EXP_PER_DEV + HALF, HALF, right, 2)
        wb_l = wcopy(right * EXP_PER_DEV, HALF, left, 3)
        wb_r.start()
        wb_l.start()

        compute_block(left)
        compute_block(right)

        wb_r.wait_recv()
        wb_l.wait_recv()
        compute_block(diag)

        for r in (wa_r, wa_l, wb_r, wb_l, c_to_r, c_to_l, c_to_d):
            r.wait_send()

    return pl.pallas_call(
        body,
        out_shape=jax.ShapeDtypeStruct((n_tok, d_ff), jnp.float32),
        in_specs=[
            pl.BlockSpec(memory_space=pltpu.VMEM),
            pl.BlockSpec(memory_space=pltpu.VMEM),
            pl.BlockSpec(memory_space=pltpu.VMEM),
        ],
        out_specs=pl.BlockSpec(memory_space=pltpu.VMEM),
        scratch_shapes=[
            pltpu.VMEM((N_EXP, d_model, d_ff), jnp.bfloat16),
            pltpu.VMEM((N_DEV, 1, LANES), jnp.int32),
            pltpu.SemaphoreType.DMA((4,)),
            pltpu.SemaphoreType.DMA((4,)),
            pltpu.SemaphoreType.DMA((3,)),
            pltpu.SemaphoreType.DMA((3,)),
        ],
        compiler_params=pltpu.CompilerParams(
            collective_id=0,
            vmem_limit_bytes=100 * 1024 * 1024,
        ),
    )(x.astype(jnp.bfloat16), route_idx, expert_W.astype(jnp.bfloat16))


# device time: 101669 ns/iter; 2.0290x vs baseline; 1.0765x over previous
import jax
import jax.numpy as jnp
from jax import lax
from jax.experimental import pallas as pl
from jax.experimental.pallas import tpu as pltpu

N_DEV = 4
N_EXP = 16
EXP_PER_DEV = N_EXP // N_DEV
HALF = EXP_PER_DEV // 2
CAP = 409
LANES = 128


def kernel(x, router_W, route_idx, expert_W):
    n_tok, d_model = x.shape
    _, _, d_ff = expert_W.shape

    def body(x_ref, route_ref, ew_ref, out_ref,
             gw_ref, gcnt_ref, w_send, w_recv, c_send, c_recv):
        my = lax.axis_index("i")
        left = lax.rem(my + N_DEV - 1, N_DEV)
        right = lax.rem(my + 1, N_DEV)
        diag = lax.rem(my + 2, N_DEV)

        eloc = lax.broadcasted_iota(jnp.int32, (n_tok, N_EXP), 1)
        m_loc = (route_ref[...] == eloc).astype(jnp.int32)
        cnt = jnp.sum(m_loc, axis=0, keepdims=True)

        gw_ref[pl.ds(my * EXP_PER_DEV, EXP_PER_DEV)] = (
            ew_ref[...].astype(jnp.bfloat16))
        gcnt_ref[pl.ds(my, 1)] = jnp.zeros((1, 1, LANES), jnp.int32)
        gcnt_ref[pl.ds(my, 1), 0:1, 0:N_EXP] = cnt.reshape(1, 1, N_EXP)

        barrier = pltpu.get_barrier_semaphore()
        for nbr in (left, right, diag):
            pl.semaphore_signal(barrier, inc=1, device_id=(nbr,),
                                device_id_type=pl.DeviceIdType.MESH)
        pl.semaphore_wait(barrier, N_DEV - 1)

        def wcopy(src_lo, n_exp, target, ch):
            return pltpu.make_async_remote_copy(
                src_ref=gw_ref.at[pl.ds(src_lo, n_exp)],
                dst_ref=gw_ref.at[pl.ds(src_lo, n_exp)],
                send_sem=w_send.at[ch],
                recv_sem=w_recv.at[ch],
                device_id=(target,),
                device_id_type=pl.DeviceIdType.MESH,
            )

        def ccopy(target, ch):
            return pltpu.make_async_remote_copy(
                src_ref=gcnt_ref.at[pl.ds(my, 1)],
                dst_ref=gcnt_ref.at[pl.ds(my, 1)],
                send_sem=c_send.at[ch],
                recv_sem=c_recv.at[ch],
                device_id=(target,),
                device_id_type=pl.DeviceIdType.MESH,
            )

        c_to_r = ccopy(right, 0)
        c_to_l = ccopy(left, 1)
        c_to_d = ccopy(diag, 2)
        c_to_r.start()
        c_to_l.start()
        c_to_d.start()

        wa_r = wcopy(my * EXP_PER_DEV, EXP_PER_DEV, right, 0)
        wa_l = wcopy(my * EXP_PER_DEV, EXP_PER_DEV, left, 1)
        wa_r.start()
        wa_l.start()

        c = m_loc
        s = 1
        while s < n_tok:
            c = c + jnp.concatenate(
                [jnp.zeros((s, N_EXP), jnp.int32), c[:-s, :]], axis=0)
            s *= 2
        rank_excl = c - m_loc

        c_to_r.wait_recv()
        c_to_l.wait_recv()
        c_to_d.wait_recv()
        gcnt = gcnt_ref[...]
        j = lax.broadcasted_iota(jnp.int32, (N_DEV, 1, LANES), 0)
        prefix = jnp.sum(jnp.where(j < my, gcnt, 0), axis=0)
        prefix16 = prefix[0:1, 0:N_EXP]

        kept = (m_loc > 0) & ((prefix16 + rank_excl) < CAP)
        kept_tok = jnp.sum(kept, axis=1, keepdims=True) > 0

        x_bf = x_ref[...].astype(jnp.bfloat16)
        route = route_ref[...]

        def block_partial(blk):
            acc = None
            for k in range(EXP_PER_DEV):
                e = blk * EXP_PER_DEV + k
                sel = (route == e) & kept_tok
                xe = jnp.where(sel, x_bf, jnp.bfloat16(0))
                w = gw_ref[pl.ds(e, 1)][0]
                p = jnp.dot(xe, w, preferred_element_type=jnp.float32)
                acc = p if acc is None else acc + p
            return acc


        out_ref[...] = block_partial(my)

        wa_r.wait_recv()
        wa_l.wait_recv()
        wb_r = wcopy(left * EXP_PER_DEV + HALF, HALF, right, 2)
        wb_l = wcopy(right * EXP_PER_DEV, HALF, left, 3)
        wb_r.start()
        wb_l.start()

        out_ref[...] += block_partial(left) + block_partial(right)

        wb_r.wait_recv()
        wb_l.wait_recv()
        out_ref[...] += block_partial(diag)

        for r in (wa_r, wa_l, wb_r, wb_l, c_to_r, c_to_l, c_to_d):
            r.wait_send()

    return pl.pallas_call(
        body,
        out_shape=jax.ShapeDtypeStruct((n_tok, d_ff), jnp.float32),
        in_specs=[
            pl.BlockSpec(memory_space=pltpu.VMEM),
            pl.BlockSpec(memory_space=pltpu.VMEM),
            pl.BlockSpec(memory_space=pltpu.VMEM),
        ],
        out_specs=pl.BlockSpec(memory_space=pltpu.VMEM),
        scratch_shapes=[
            pltpu.VMEM((N_EXP, d_model, d_ff), jnp.bfloat16),
            pltpu.VMEM((N_DEV, 1, LANES), jnp.int32),
            pltpu.SemaphoreType.DMA((4,)),
            pltpu.SemaphoreType.DMA((4,)),
            pltpu.SemaphoreType.DMA((3,)),
            pltpu.SemaphoreType.DMA((3,)),
        ],
        compiler_params=pltpu.CompilerParams(
            collective_id=0,
            vmem_limit_bytes=100 * 1024 * 1024,
        ),
    )(x, route_idx, expert_W)


# device time: 95229 ns/iter; 2.1663x vs baseline; 1.0676x over previous
import jax
import jax.numpy as jnp
from jax import lax
from jax.experimental import pallas as pl
from jax.experimental.pallas import tpu as pltpu

N_DEV = 4
N_EXP = 16
EXP_PER_DEV = N_EXP // N_DEV
HALF = EXP_PER_DEV // 2
CAP = 409
LANES = 128

A1_FROM_L, A2_FROM_L, A1_FROM_R, A2_FROM_R = 0, 1, 2, 3
B1_FROM_L, B2_FROM_L, B1_FROM_R, B2_FROM_R = 4, 5, 6, 7
N_WCH = 8


def kernel(x, router_W, route_idx, expert_W):
    n_tok, d_model = x.shape
    _, _, d_ff = expert_W.shape

    def body(x_ref, route_ref, ew_ref, out_ref,
             gw_ref, gcnt_ref, w_send, w_recv, c_send, c_recv):
        my = lax.axis_index("i")
        left = lax.rem(my + N_DEV - 1, N_DEV)
        right = lax.rem(my + 1, N_DEV)
        diag = lax.rem(my + 2, N_DEV)

        eloc = lax.broadcasted_iota(jnp.int32, (n_tok, N_EXP), 1)
        m_loc = (route_ref[...] == eloc).astype(jnp.int32)
        cnt = jnp.sum(m_loc, axis=0, keepdims=True)

        gw_ref[pl.ds(my * EXP_PER_DEV, EXP_PER_DEV)] = (
            ew_ref[...].astype(jnp.bfloat16))
        gcnt_ref[pl.ds(my, 1)] = jnp.zeros((1, 1, LANES), jnp.int32)
        gcnt_ref[pl.ds(my, 1), 0:1, 0:N_EXP] = cnt.reshape(1, 1, N_EXP)

        barrier = pltpu.get_barrier_semaphore()
        for nbr in (left, right, diag):
            pl.semaphore_signal(barrier, inc=1, device_id=(nbr,),
                                device_id_type=pl.DeviceIdType.MESH)
        pl.semaphore_wait(barrier, N_DEV - 1)

        def wcopy(src_lo, n_exp, target, ch):
            return pltpu.make_async_remote_copy(
                src_ref=gw_ref.at[pl.ds(src_lo, n_exp)],
                dst_ref=gw_ref.at[pl.ds(src_lo, n_exp)],
                send_sem=w_send.at[ch],
                recv_sem=w_recv.at[ch],
                device_id=(target,),
                device_id_type=pl.DeviceIdType.MESH,
            )

        def ccopy(target, ch):
            return pltpu.make_async_remote_copy(
                src_ref=gcnt_ref.at[pl.ds(my, 1)],
                dst_ref=gcnt_ref.at[pl.ds(my, 1)],
                send_sem=c_send.at[ch],
                recv_sem=c_recv.at[ch],
                device_id=(target,),
                device_id_type=pl.DeviceIdType.MESH,
            )

        c_to_r = ccopy(right, 0)
        c_to_l = ccopy(left, 1)
        c_to_d = ccopy(diag, 2)
        c_to_r.start()
        c_to_l.start()
        c_to_d.start()

        wa_r1 = wcopy(my * EXP_PER_DEV + HALF, HALF, right, A1_FROM_L)
        wa_l1 = wcopy(my * EXP_PER_DEV, HALF, left, A1_FROM_R)
        wa_r2 = wcopy(my * EXP_PER_DEV, HALF, right, A2_FROM_L)
        wa_l2 = wcopy(my * EXP_PER_DEV + HALF, HALF, left, A2_FROM_R)
        wa_r1.start()
        wa_l1.start()
        wa_r2.start()
        wa_l2.start()

        c = m_loc
        s = 1
        while s < n_tok:
            c = c + jnp.concatenate(
                [jnp.zeros((s, N_EXP), jnp.int32), c[:-s, :]], axis=0)
            s *= 2
        rank_excl = c - m_loc

        c_to_r.wait_recv()
        c_to_l.wait_recv()
        c_to_d.wait_recv()
        gcnt = gcnt_ref[...]
        j = lax.broadcasted_iota(jnp.int32, (N_DEV, 1, LANES), 0)
        prefix = jnp.sum(jnp.where(j < my, gcnt, 0), axis=0)
        prefix16 = prefix[0:1, 0:N_EXP]

        kept = (m_loc > 0) & ((prefix16 + rank_excl) < CAP)
        kept_tok = jnp.sum(kept, axis=1, keepdims=True) > 0

        x_bf = x_ref[...].astype(jnp.bfloat16)
        route = route_ref[...]

        def partial(blk, ks):
            acc = None
            for k in ks:
                e = blk * EXP_PER_DEV + k
                sel = (route == e) & kept_tok
                xe = jnp.where(sel, x_bf, jnp.bfloat16(0))
                w = gw_ref[pl.ds(e, 1)][0]
                p = jnp.dot(xe, w, preferred_element_type=jnp.float32)
                acc = p if acc is None else acc + p
            return acc

        out_ref[...] = partial(my, range(EXP_PER_DEV))

        wa_r1.wait_recv()
        wb_r1 = wcopy(left * EXP_PER_DEV + HALF, 1, right, B1_FROM_L)
        wb_r2 = wcopy(left * EXP_PER_DEV + HALF + 1, 1, right, B2_FROM_L)
        wb_r1.start()
        wb_r2.start()
        wa_l1.wait_recv()
        wb_l1 = wcopy(right * EXP_PER_DEV, 1, left, B1_FROM_R)
        wb_l2 = wcopy(right * EXP_PER_DEV + 1, 1, left, B2_FROM_R)
        wb_l1.start()
        wb_l2.start()

        out_ref[...] += partial(left, (HALF, HALF + 1))
        out_ref[...] += partial(right, (0, 1))

        wa_r2.wait_recv()
        wa_l2.wait_recv()
        out_ref[...] += partial(left, (0, 1))
        out_ref[...] += partial(right, (HALF, HALF + 1))

        wb_r1.wait_recv()
        wb_l1.wait_recv()
        out_ref[...] += partial(diag, (HALF, 0))
        wb_r2.wait_recv()
        wb_l2.wait_recv()
        out_ref[...] += partial(diag, (HALF + 1, 1))

        for r in (wa_r1, wa_r2, wa_l1, wa_l2,
                  wb_r1, wb_r2, wb_l1, wb_l2,
                  c_to_r, c_to_l, c_to_d):
            r.wait_send()

    return pl.pallas_call(
        body,
        out_shape=jax.ShapeDtypeStruct((n_tok, d_ff), jnp.float32),
        in_specs=[
            pl.BlockSpec(memory_space=pltpu.VMEM),
            pl.BlockSpec(memory_space=pltpu.VMEM),
            pl.BlockSpec(memory_space=pltpu.VMEM),
        ],
        out_specs=pl.BlockSpec(memory_space=pltpu.VMEM),
        scratch_shapes=[
            pltpu.VMEM((N_EXP, d_model, d_ff), jnp.bfloat16),
            pltpu.VMEM((N_DEV, 1, LANES), jnp.int32),
            pltpu.SemaphoreType.DMA((N_WCH,)),
            pltpu.SemaphoreType.DMA((N_WCH,)),
            pltpu.SemaphoreType.DMA((3,)),
            pltpu.SemaphoreType.DMA((3,)),
        ],
        compiler_params=pltpu.CompilerParams(
            collective_id=0,
            vmem_limit_bytes=100 * 1024 * 1024,
        ),
    )(x, route_idx, expert_W)
